# Initial kernel scaffold; baseline (speedup 1.0000x reference)
import jax
import jax.numpy as jnp
from jax.experimental import pallas as pl


def kernel(x, edge_index, Wl1, bl1, Wr1, Wl2, bl2, Wr2):
    raise NotImplementedError("write your pallas kernel here")


# speedup vs baseline: 3.1001x; 3.1001x over previous
import dataclasses
import functools

import jax
import jax.numpy as jnp
from jax import lax
from jax.experimental import pallas as pl
from jax.experimental.pallas import tpu as pltpu
from jax.experimental.pallas import tpu_sc as plsc

N = 10000
E = 320000
D = 128

BASEN = 5000
ACCR = 5120
CROWS = 48
CH = 96
NCHUNK = 209
PER_TILE = CH * NCHUNK
EPAD = PER_TILE * 16

_mesh = plsc.VectorSubcoreMesh(core_axis_name="c", subcore_axis_name="s")
_cp = pltpu.CompilerParams()
if "needs_layout_passes" in pltpu.CompilerParams.__dataclass_fields__:
    _cp = dataclasses.replace(_cp, needs_layout_passes=False)


def _sc_body(with_counts, y_hbm, src_hbm, dst_hbm, out_hbm, *refs):
    if with_counts:
        (srcv, dstv, dstv2, cidx, buf, zacc, ctile, acc, sem) = refs
    else:
        (srcv, dstv, dstv2, buf, zacc, acc, sem) = refs
    cid = lax.axis_index("c")
    sid = lax.axis_index("s")
    row0 = sid * (ACCR // 16)
    base = cid * BASEN

    @pl.loop(0, 8)
    def _(i):
        @pl.loop(0, D, step=16)
        def _(j):
            zacc[i, pl.ds(j, 16)] = jnp.zeros((16,), jnp.float32)

    if with_counts:
        @pl.loop(0, CROWS)
        def _(i):
            @pl.loop(0, D, step=16)
            def _(j):
                ctile[i, pl.ds(j, 16)] = jnp.zeros((16,), jnp.float32)

        @pl.loop(0, CROWS, step=16)
        def _(i):
            cidx[pl.ds(i, 16)] = lax.iota(jnp.int32, 16) + (BASEN + i)

    @pl.loop(0, ACCR // 16, step=8)
    def _(k):
        pltpu.sync_copy(zacc, acc.at[pl.ds(row0 + k, 8)])

    plsc.subcore_barrier()

    ebase = sid * PER_TILE

    @pl.loop(0, PER_TILE, step=CH)
    def _(j):
        off = ebase + j
        pltpu.sync_copy(src_hbm.at[pl.ds(off, CH)], srcv)
        pltpu.sync_copy(dst_hbm.at[pl.ds(off, CH)], dstv)
        pltpu.async_copy(y_hbm.at[srcv], buf, sem).wait()

        @pl.loop(0, CH, step=16)
        def _(g):
            d = dstv[pl.ds(g, 16)]
            t = d - base
            inb = (t >= 0) & (t < BASEN)
            jr = (BASEN + CROWS) + lax.iota(jnp.int32, 16) + (g % 64)
            t2 = jnp.where(inb, t, jr)
            dstv2[pl.ds(g, 16)] = t2
            if with_counts:
                ts = jnp.where(inb, t, 0)
                row = lax.shift_right_logical(ts, 7)
                col = ts & 127
                plsc.addupdate_scatter(ctile, [row, col],
                                       jnp.ones((16,), jnp.float32), mask=inb)

        pltpu.sync_copy(buf, acc.at[dstv2], add=True)

    if with_counts:
        pltpu.sync_copy(ctile, acc.at[cidx], add=True)
    plsc.subcore_barrier()
    pltpu.sync_copy(acc.at[pl.ds(row0, ACCR // 16)],
                    out_hbm.at[cid, pl.ds(row0, ACCR // 16)])


@jax.jit
def _sc_pass_cnt(y, src, dst):
    f = pl.kernel(
        functools.partial(_sc_body, True),
        out_type=jax.ShapeDtypeStruct((2, ACCR, D), jnp.float32),
        mesh=_mesh,
        scratch_types=[
            pltpu.VMEM((CH,), jnp.int32),
            pltpu.VMEM((CH,), jnp.int32),
            pltpu.VMEM((CH,), jnp.int32),
            pltpu.VMEM((CROWS,), jnp.int32),
            pltpu.VMEM((CH, D), jnp.float32),
            pltpu.VMEM((8, D), jnp.float32),
            pltpu.VMEM((CROWS, D), jnp.float32),
            pltpu.VMEM_SHARED((ACCR, D), jnp.float32),
            pltpu.SemaphoreType.DMA,
        ],
        compiler_params=_cp,
    )
    return f(y, src, dst)


@jax.jit
def _sc_pass(y, src, dst):
    f = pl.kernel(
        functools.partial(_sc_body, False),
        out_type=jax.ShapeDtypeStruct((2, ACCR, D), jnp.float32),
        mesh=_mesh,
        scratch_types=[
            pltpu.VMEM((CH,), jnp.int32),
            pltpu.VMEM((CH,), jnp.int32),
            pltpu.VMEM((CH,), jnp.int32),
            pltpu.VMEM((CH, D), jnp.float32),
            pltpu.VMEM((8, D), jnp.float32),
            pltpu.VMEM_SHARED((ACCR, D), jnp.float32),
            pltpu.SemaphoreType.DMA,
        ],
        compiler_params=_cp,
    )
    return f(y, src, dst)


def _mm_pre_body(h_ref, wl_ref, wr_ref, b_ref, y_ref, r_ref):
    h = h_ref[...]
    dn = (((1,), (1,)), ((), ()))
    y_ref[...] = lax.dot_general(h, wl_ref[...], dn,
                                 preferred_element_type=jnp.float32)
    r_ref[...] = lax.dot_general(h, wr_ref[...], dn,
                                 preferred_element_type=jnp.float32) + b_ref[...]


def _mm_pre(h, wl, wr, b):
    nb = 10
    rows = N // nb
    return pl.pallas_call(
        _mm_pre_body,
        grid=(nb,),
        in_specs=[
            pl.BlockSpec((rows, D), lambda i: (i, 0)),
            pl.BlockSpec((D, D), lambda i: (0, 0)),
            pl.BlockSpec((D, D), lambda i: (0, 0)),
            pl.BlockSpec((1, D), lambda i: (0, 0)),
        ],
        out_specs=[
            pl.BlockSpec((rows, D), lambda i: (i, 0)),
            pl.BlockSpec((rows, D), lambda i: (i, 0)),
        ],
        out_shape=[
            jax.ShapeDtypeStruct((N, D), jnp.float32),
            jax.ShapeDtypeStruct((N, D), jnp.float32),
        ],
    )(h, wl, wr, b)


def _combine1_body(agg_ref, cnt_ref, r_ref, wl_ref, wr_ref, b_ref,
                   y_ref, r2_ref):
    mean = agg_ref[...] / jnp.maximum(cnt_ref[...], 1.0)
    h = mean + r_ref[...]
    h = jnp.where(h >= 0, h, 0.01 * h)
    dn = (((1,), (1,)), ((), ()))
    y_ref[...] = lax.dot_general(h, wl_ref[...], dn,
                                 preferred_element_type=jnp.float32)
    r2_ref[...] = lax.dot_general(h, wr_ref[...], dn,
                                  preferred_element_type=jnp.float32) + b_ref[...]


def _combine1(agg, cnt, r, wl, wr, b):
    nb = 10
    rows = N // nb
    return pl.pallas_call(
        _combine1_body,
        grid=(nb,),
        in_specs=[
            pl.BlockSpec((rows, D), lambda i: (i, 0)),
            pl.BlockSpec((rows, 1), lambda i: (i, 0)),
            pl.BlockSpec((rows, D), lambda i: (i, 0)),
            pl.BlockSpec((D, D), lambda i: (0, 0)),
            pl.BlockSpec((D, D), lambda i: (0, 0)),
            pl.BlockSpec((1, D), lambda i: (0, 0)),
        ],
        out_specs=[
            pl.BlockSpec((rows, D), lambda i: (i, 0)),
            pl.BlockSpec((rows, D), lambda i: (i, 0)),
        ],
        out_shape=[
            jax.ShapeDtypeStruct((N, D), jnp.float32),
            jax.ShapeDtypeStruct((N, D), jnp.float32),
        ],
    )(agg, cnt, r, wl, wr, b)


def _final_body(agg_ref, cnt_ref, r_ref, o_ref):
    mean = agg_ref[...] / jnp.maximum(cnt_ref[...], 1.0)
    h = mean + r_ref[...]
    m = jnp.max(h, axis=1, keepdims=True)
    ex = jnp.exp(h - m)
    s = jnp.sum(ex, axis=1, keepdims=True)
    o_ref[...] = h - m - jnp.log(s)


def _final(agg, cnt, r):
    nb = 10
    rows = N // nb
    return pl.pallas_call(
        _final_body,
        grid=(nb,),
        in_specs=[
            pl.BlockSpec((rows, D), lambda i: (i, 0)),
            pl.BlockSpec((rows, 1), lambda i: (i, 0)),
            pl.BlockSpec((rows, D), lambda i: (i, 0)),
        ],
        out_specs=pl.BlockSpec((rows, D), lambda i: (i, 0)),
        out_shape=jax.ShapeDtypeStruct((N, D), jnp.float32),
    )(agg, cnt, r)


def _assemble(acc):
    agg = jnp.concatenate([acc[0, :BASEN], acc[1, :BASEN]], axis=0)
    c = acc[:, BASEN:BASEN + CROWS, :].reshape(2, CROWS * D)[:, :BASEN]
    cnt = c.reshape(2 * BASEN, 1)
    return agg, cnt


def kernel(x, edge_index, Wl1, bl1, Wr1, Wl2, bl2, Wr2):
    src = edge_index[0]
    dst = edge_index[1]
    pad = EPAD - E
    src_p = jnp.concatenate([src, jnp.zeros((pad,), jnp.int32)])
    dst_p = jnp.concatenate([dst, jnp.full((pad,), N, jnp.int32)])
    b1 = bl1.reshape(1, D)
    b2 = bl2.reshape(1, D)

    y1, r1 = _mm_pre(x, Wl1, Wr1, b1)
    acc1 = _sc_pass_cnt(y1, src_p, dst_p)
    agg1, cnt = _assemble(acc1)
    y2, r2 = _combine1(agg1, cnt, r1, Wl2, Wr2, b2)
    acc2 = _sc_pass(y2, src_p, dst_p)
    agg2 = jnp.concatenate([acc2[0, :BASEN], acc2[1, :BASEN]], axis=0)
    out = _final(agg2, cnt, r2)
    return out


# speedup vs baseline: 3.6750x; 1.1854x over previous
import dataclasses
import functools

import jax
import jax.numpy as jnp
from jax import lax
from jax.experimental import pallas as pl
from jax.experimental.pallas import tpu as pltpu
from jax.experimental.pallas import tpu_sc as plsc

N = 10000
E = 320000
D = 128

BASEN = 5000
ACCR = 5120
CROWS = 48
CH = 96
NCHUNK = 210
NPAIR = NCHUNK // 2
PER_TILE = CH * NCHUNK
EPAD = PER_TILE * 16

_mesh = plsc.VectorSubcoreMesh(core_axis_name="c", subcore_axis_name="s")
_cp = pltpu.CompilerParams()
if "needs_layout_passes" in pltpu.CompilerParams.__dataclass_fields__:
    _cp = dataclasses.replace(_cp, needs_layout_passes=False)


def _sc_body(with_counts, y_hbm, src_hbm, dst_hbm, out_hbm, *refs):
    if with_counts:
        (srcv0, dstv0, dstw0, srcv1, dstv1, dstw1, cidx,
         buf0, buf1, zacc, ctile, acc, sem0, sem1) = refs
    else:
        (srcv0, dstv0, dstw0, srcv1, dstv1, dstw1,
         buf0, buf1, zacc, acc, sem0, sem1) = refs
    cid = lax.axis_index("c")
    sid = lax.axis_index("s")
    row0 = sid * (ACCR // 16)
    base = cid * BASEN

    @pl.loop(0, 4)
    def _(i):
        @pl.loop(0, D, step=16)
        def _(j):
            zacc[i, pl.ds(j, 16)] = jnp.zeros((16,), jnp.float32)

    if with_counts:
        @pl.loop(0, CROWS)
        def _(i):
            @pl.loop(0, D, step=16)
            def _(j):
                ctile[i, pl.ds(j, 16)] = jnp.zeros((16,), jnp.float32)

        @pl.loop(0, CROWS, step=16)
        def _(i):
            cidx[pl.ds(i, 16)] = lax.iota(jnp.int32, 16) + (BASEN + i)

    @pl.loop(0, ACCR // 16, step=4)
    def _(k):
        pltpu.sync_copy(zacc, acc.at[pl.ds(row0 + k, 4)])

    plsc.subcore_barrier()

    ebase = sid * PER_TILE

    def load_and_start(off, srcv, dstv, sem, buf):
        pltpu.sync_copy(src_hbm.at[pl.ds(off, CH)], srcv)
        pltpu.sync_copy(dst_hbm.at[pl.ds(off, CH)], dstv)
        pltpu.async_copy(y_hbm.at[srcv], buf, sem)

    def transform(dstv, dstw):
        @pl.loop(0, CH, step=16)
        def _(g):
            d = dstv[pl.ds(g, 16)]
            t = d - base
            inb = (t >= 0) & (t < BASEN)
            jr = (BASEN + CROWS) + lax.iota(jnp.int32, 16) + (g % 64)
            t2 = jnp.where(inb, t, jr)
            dstw[pl.ds(g, 16)] = t2
            if with_counts:
                ts = jnp.where(inb, t, 0)
                row = lax.shift_right_logical(ts, 7)
                col = ts & 127
                plsc.addupdate_scatter(ctile, [row, col],
                                       jnp.ones((16,), jnp.float32), mask=inb)

    load_and_start(ebase, srcv0, dstv0, sem0, buf0)
    transform(dstv0, dstw0)

    @pl.loop(0, NPAIR)
    def _(k):
        j = ebase + k * (2 * CH)
        load_and_start(j + CH, srcv1, dstv1, sem1, buf1)
        transform(dstv1, dstw1)
        pltpu.make_async_copy(y_hbm.at[srcv0], buf0, sem0).wait()
        pltpu.sync_copy(buf0, acc.at[dstw0], add=True)

        @pl.when(k != NPAIR - 1)
        def _():
            load_and_start(j + 2 * CH, srcv0, dstv0, sem0, buf0)
            transform(dstv0, dstw0)

        pltpu.make_async_copy(y_hbm.at[srcv1], buf1, sem1).wait()
        pltpu.sync_copy(buf1, acc.at[dstw1], add=True)

    if with_counts:
        pltpu.sync_copy(ctile, acc.at[cidx], add=True)
    plsc.subcore_barrier()
    pltpu.sync_copy(acc.at[pl.ds(row0, ACCR // 16)],
                    out_hbm.at[cid, pl.ds(row0, ACCR // 16)])


def _sc_scratch(with_counts):
    s = [
        pltpu.VMEM((CH,), jnp.int32),
        pltpu.VMEM((CH,), jnp.int32),
        pltpu.VMEM((CH,), jnp.int32),
        pltpu.VMEM((CH,), jnp.int32),
        pltpu.VMEM((CH,), jnp.int32),
        pltpu.VMEM((CH,), jnp.int32),
    ]
    if with_counts:
        s.append(pltpu.VMEM((CROWS,), jnp.int32))
    s += [
        pltpu.VMEM((CH, D), jnp.float32),
        pltpu.VMEM((CH, D), jnp.float32),
        pltpu.VMEM((4, D), jnp.float32),
    ]
    if with_counts:
        s.append(pltpu.VMEM((CROWS, D), jnp.float32))
    s += [
        pltpu.VMEM_SHARED((ACCR, D), jnp.float32),
        pltpu.SemaphoreType.DMA,
        pltpu.SemaphoreType.DMA,
    ]
    return s


@jax.jit
def _sc_pass_cnt(y, src, dst):
    f = pl.kernel(
        functools.partial(_sc_body, True),
        out_type=jax.ShapeDtypeStruct((2, ACCR, D), jnp.float32),
        mesh=_mesh,
        scratch_types=_sc_scratch(True),
        compiler_params=_cp,
    )
    return f(y, src, dst)


@jax.jit
def _sc_pass(y, src, dst):
    f = pl.kernel(
        functools.partial(_sc_body, False),
        out_type=jax.ShapeDtypeStruct((2, ACCR, D), jnp.float32),
        mesh=_mesh,
        scratch_types=_sc_scratch(False),
        compiler_params=_cp,
    )
    return f(y, src, dst)


def _mm_pre_body(h_ref, wl_ref, wr_ref, b_ref, y_ref, r_ref):
    h = h_ref[...]
    dn = (((1,), (1,)), ((), ()))
    y_ref[...] = lax.dot_general(h, wl_ref[...], dn,
                                 preferred_element_type=jnp.float32)
    r_ref[...] = lax.dot_general(h, wr_ref[...], dn,
                                 preferred_element_type=jnp.float32) + b_ref[...]


def _mm_pre(h, wl, wr, b):
    nb = 10
    rows = N // nb
    return pl.pallas_call(
        _mm_pre_body,
        grid=(nb,),
        in_specs=[
            pl.BlockSpec((rows, D), lambda i: (i, 0)),
            pl.BlockSpec((D, D), lambda i: (0, 0)),
            pl.BlockSpec((D, D), lambda i: (0, 0)),
            pl.BlockSpec((1, D), lambda i: (0, 0)),
        ],
        out_specs=[
            pl.BlockSpec((rows, D), lambda i: (i, 0)),
            pl.BlockSpec((rows, D), lambda i: (i, 0)),
        ],
        out_shape=[
            jax.ShapeDtypeStruct((N, D), jnp.float32),
            jax.ShapeDtypeStruct((N, D), jnp.float32),
        ],
    )(h, wl, wr, b)


def _combine1_body(agg_ref, cnt_ref, r_ref, wl_ref, wr_ref, b_ref,
                   y_ref, r2_ref):
    mean = agg_ref[...] / jnp.maximum(cnt_ref[...], 1.0)
    h = mean + r_ref[...]
    h = jnp.where(h >= 0, h, 0.01 * h)
    dn = (((1,), (1,)), ((), ()))
    y_ref[...] = lax.dot_general(h, wl_ref[...], dn,
                                 preferred_element_type=jnp.float32)
    r2_ref[...] = lax.dot_general(h, wr_ref[...], dn,
                                  preferred_element_type=jnp.float32) + b_ref[...]


def _combine1(agg, cnt, r, wl, wr, b):
    nb = 10
    rows = N // nb
    return pl.pallas_call(
        _combine1_body,
        grid=(nb,),
        in_specs=[
            pl.BlockSpec((rows, D), lambda i: (i, 0)),
            pl.BlockSpec((rows, 1), lambda i: (i, 0)),
            pl.BlockSpec((rows, D), lambda i: (i, 0)),
            pl.BlockSpec((D, D), lambda i: (0, 0)),
            pl.BlockSpec((D, D), lambda i: (0, 0)),
            pl.BlockSpec((1, D), lambda i: (0, 0)),
        ],
        out_specs=[
            pl.BlockSpec((rows, D), lambda i: (i, 0)),
            pl.BlockSpec((rows, D), lambda i: (i, 0)),
        ],
        out_shape=[
            jax.ShapeDtypeStruct((N, D), jnp.float32),
            jax.ShapeDtypeStruct((N, D), jnp.float32),
        ],
    )(agg, cnt, r, wl, wr, b)


def _final_body(agg_ref, cnt_ref, r_ref, o_ref):
    mean = agg_ref[...] / jnp.maximum(cnt_ref[...], 1.0)
    h = mean + r_ref[...]
    m = jnp.max(h, axis=1, keepdims=True)
    ex = jnp.exp(h - m)
    s = jnp.sum(ex, axis=1, keepdims=True)
    o_ref[...] = h - m - jnp.log(s)


def _final(agg, cnt, r):
    nb = 10
    rows = N // nb
    return pl.pallas_call(
        _final_body,
        grid=(nb,),
        in_specs=[
            pl.BlockSpec((rows, D), lambda i: (i, 0)),
            pl.BlockSpec((rows, 1), lambda i: (i, 0)),
            pl.BlockSpec((rows, D), lambda i: (i, 0)),
        ],
        out_specs=pl.BlockSpec((rows, D), lambda i: (i, 0)),
        out_shape=jax.ShapeDtypeStruct((N, D), jnp.float32),
    )(agg, cnt, r)


def _assemble(acc):
    agg = jnp.concatenate([acc[0, :BASEN], acc[1, :BASEN]], axis=0)
    c = acc[:, BASEN:BASEN + CROWS, :].reshape(2, CROWS * D)[:, :BASEN]
    cnt = c.reshape(2 * BASEN, 1)
    return agg, cnt


def kernel(x, edge_index, Wl1, bl1, Wr1, Wl2, bl2, Wr2):
    src = edge_index[0]
    dst = edge_index[1]
    pad = EPAD - E
    src_p = jnp.concatenate([src, jnp.zeros((pad,), jnp.int32)])
    dst_p = jnp.concatenate([dst, jnp.full((pad,), N, jnp.int32)])
    b1 = bl1.reshape(1, D)
    b2 = bl2.reshape(1, D)

    y1, r1 = _mm_pre(x, Wl1, Wr1, b1)
    acc1 = _sc_pass_cnt(y1, src_p, dst_p)
    agg1, cnt = _assemble(acc1)
    y2, r2 = _combine1(agg1, cnt, r1, Wl2, Wr2, b2)
    acc2 = _sc_pass(y2, src_p, dst_p)
    agg2 = jnp.concatenate([acc2[0, :BASEN], acc2[1, :BASEN]], axis=0)
    out = _final(agg2, cnt, r2)
    return out


# speedup vs baseline: 4.0675x; 1.1068x over previous
import dataclasses
import functools

import jax
import jax.numpy as jnp
from jax import lax
from jax.experimental import pallas as pl
from jax.experimental.pallas import tpu as pltpu
from jax.experimental.pallas import tpu_sc as plsc

N = 10000
E = 320000
D = 128

BASEN = 5000
ACCR = 5120
CROWS = 48
CH = 96
NCHUNK = 210
NPAIR = NCHUNK // 2
PER_TILE = CH * NCHUNK
EPAD = PER_TILE * 16

_mesh = plsc.VectorSubcoreMesh(core_axis_name="c", subcore_axis_name="s")
_cp = pltpu.CompilerParams()
if "needs_layout_passes" in pltpu.CompilerParams.__dataclass_fields__:
    _cp = dataclasses.replace(_cp, needs_layout_passes=False)


def _sc_body(with_counts, y_hbm, src_hbm, dst_hbm, out_hbm, *refs):
    if with_counts:
        (srcv0, dstv0, dstw0, srcv1, dstv1, dstw1, cidx,
         buf0, buf1, zacc, ctile, acc, sem0, sem1, semi0, semi1) = refs
    else:
        (srcv0, dstv0, dstw0, srcv1, dstv1, dstw1,
         buf0, buf1, zacc, acc, sem0, sem1, semi0, semi1) = refs
    cid = lax.axis_index("c")
    sid = lax.axis_index("s")
    row0 = sid * (ACCR // 16)
    base = cid * BASEN

    @pl.loop(0, 4)
    def _(i):
        @pl.loop(0, D, step=16)
        def _(j):
            zacc[i, pl.ds(j, 16)] = jnp.zeros((16,), jnp.float32)

    if with_counts:
        @pl.loop(0, CROWS)
        def _(i):
            @pl.loop(0, D, step=16)
            def _(j):
                ctile[i, pl.ds(j, 16)] = jnp.zeros((16,), jnp.float32)

        @pl.loop(0, CROWS, step=16)
        def _(i):
            cidx[pl.ds(i, 16)] = lax.iota(jnp.int32, 16) + (BASEN + i)

    @pl.loop(0, ACCR // 16, step=4)
    def _(k):
        pltpu.sync_copy(zacc, acc.at[pl.ds(row0 + k, 4)])

    plsc.subcore_barrier()

    ebase = sid * PER_TILE

    def idxload(off, srcv, dstv, semi):
        pltpu.async_copy(src_hbm.at[pl.ds(off, CH)], srcv, semi)
        pltpu.async_copy(dst_hbm.at[pl.ds(off, CH)], dstv, semi)

    def idxwait(off, srcv, dstv, semi):
        pltpu.make_async_copy(src_hbm.at[pl.ds(off, CH)], srcv, semi).wait()
        pltpu.make_async_copy(dst_hbm.at[pl.ds(off, CH)], dstv, semi).wait()

    def gstart(srcv, buf, sem):
        pltpu.async_copy(y_hbm.at[srcv], buf, sem)

    def gwait(srcv, buf, sem):
        pltpu.make_async_copy(y_hbm.at[srcv], buf, sem).wait()

    def transform(dstv, dstw):
        @pl.loop(0, CH, step=16)
        def _(g):
            d = dstv[pl.ds(g, 16)]
            t = d - base
            inb = (t >= 0) & (t < BASEN)
            jr = (BASEN + CROWS) + lax.iota(jnp.int32, 16) + (g % 64)
            t2 = jnp.where(inb, t, jr)
            dstw[pl.ds(g, 16)] = t2
            if with_counts:
                ts = jnp.where(inb, t, 0)
                row = lax.shift_right_logical(ts, 7)
                col = ts & 127
                plsc.addupdate_scatter(ctile, [row, col],
                                       jnp.ones((16,), jnp.float32), mask=inb)

    pltpu.sync_copy(src_hbm.at[pl.ds(ebase, CH)], srcv0)
    pltpu.sync_copy(dst_hbm.at[pl.ds(ebase, CH)], dstv0)
    gstart(srcv0, buf0, sem0)
    transform(dstv0, dstw0)
    idxload(ebase + CH, srcv1, dstv1, semi1)

    @pl.loop(0, NPAIR)
    def _(k):
        j = ebase + k * (2 * CH)
        idxwait(j + CH, srcv1, dstv1, semi1)
        gstart(srcv1, buf1, sem1)
        transform(dstv1, dstw1)
        gwait(srcv0, buf0, sem0)
        pltpu.sync_copy(buf0, acc.at[dstw0], add=True)

        @pl.when(k != NPAIR - 1)
        def _():
            idxload(j + 2 * CH, srcv0, dstv0, semi0)

        gwait(srcv1, buf1, sem1)
        pltpu.sync_copy(buf1, acc.at[dstw1], add=True)

        @pl.when(k != NPAIR - 1)
        def _():
            idxwait(j + 2 * CH, srcv0, dstv0, semi0)
            gstart(srcv0, buf0, sem0)
            transform(dstv0, dstw0)
            idxload(j + 3 * CH, srcv1, dstv1, semi1)

    if with_counts:
        pltpu.sync_copy(ctile, acc.at[cidx], add=True)
    plsc.subcore_barrier()
    pltpu.sync_copy(acc.at[pl.ds(row0, ACCR // 16)],
                    out_hbm.at[cid, pl.ds(row0, ACCR // 16)])


def _sc_scratch(with_counts):
    s = [
        pltpu.VMEM((CH,), jnp.int32),
        pltpu.VMEM((CH,), jnp.int32),
        pltpu.VMEM((CH,), jnp.int32),
        pltpu.VMEM((CH,), jnp.int32),
        pltpu.VMEM((CH,), jnp.int32),
        pltpu.VMEM((CH,), jnp.int32),
    ]
    if with_counts:
        s.append(pltpu.VMEM((CROWS,), jnp.int32))
    s += [
        pltpu.VMEM((CH, D), jnp.float32),
        pltpu.VMEM((CH, D), jnp.float32),
        pltpu.VMEM((4, D), jnp.float32),
    ]
    if with_counts:
        s.append(pltpu.VMEM((CROWS, D), jnp.float32))
    s += [
        pltpu.VMEM_SHARED((ACCR, D), jnp.float32),
        pltpu.SemaphoreType.DMA,
        pltpu.SemaphoreType.DMA,
        pltpu.SemaphoreType.DMA,
        pltpu.SemaphoreType.DMA,
    ]
    return s


@jax.jit
def _sc_pass_cnt(y, src, dst):
    f = pl.kernel(
        functools.partial(_sc_body, True),
        out_type=jax.ShapeDtypeStruct((2, ACCR, D), jnp.float32),
        mesh=_mesh,
        scratch_types=_sc_scratch(True),
        compiler_params=_cp,
    )
    return f(y, src, dst)


@jax.jit
def _sc_pass(y, src, dst):
    f = pl.kernel(
        functools.partial(_sc_body, False),
        out_type=jax.ShapeDtypeStruct((2, ACCR, D), jnp.float32),
        mesh=_mesh,
        scratch_types=_sc_scratch(False),
        compiler_params=_cp,
    )
    return f(y, src, dst)


def _mm_pre_body(h_ref, wl_ref, wr_ref, b_ref, y_ref, r_ref):
    h = h_ref[...]
    dn = (((1,), (1,)), ((), ()))
    y_ref[...] = lax.dot_general(h, wl_ref[...], dn,
                                 preferred_element_type=jnp.float32)
    r_ref[...] = lax.dot_general(h, wr_ref[...], dn,
                                 preferred_element_type=jnp.float32) + b_ref[...]


def _mm_pre(h, wl, wr, b):
    nb = 10
    rows = N // nb
    return pl.pallas_call(
        _mm_pre_body,
        grid=(nb,),
        in_specs=[
            pl.BlockSpec((rows, D), lambda i: (i, 0)),
            pl.BlockSpec((D, D), lambda i: (0, 0)),
            pl.BlockSpec((D, D), lambda i: (0, 0)),
            pl.BlockSpec((1, D), lambda i: (0, 0)),
        ],
        out_specs=[
            pl.BlockSpec((rows, D), lambda i: (i, 0)),
            pl.BlockSpec((rows, D), lambda i: (i, 0)),
        ],
        out_shape=[
            jax.ShapeDtypeStruct((N, D), jnp.float32),
            jax.ShapeDtypeStruct((N, D), jnp.float32),
        ],
    )(h, wl, wr, b)


def _combine1_body(agg_ref, cnt_ref, r_ref, wl_ref, wr_ref, b_ref,
                   y_ref, r2_ref):
    mean = agg_ref[...] / jnp.maximum(cnt_ref[...], 1.0)
    h = mean + r_ref[...]
    h = jnp.where(h >= 0, h, 0.01 * h)
    dn = (((1,), (1,)), ((), ()))
    y_ref[...] = lax.dot_general(h, wl_ref[...], dn,
                                 preferred_element_type=jnp.float32)
    r2_ref[...] = lax.dot_general(h, wr_ref[...], dn,
                                  preferred_element_type=jnp.float32) + b_ref[...]


def _combine1(agg, cnt, r, wl, wr, b):
    nb = 10
    rows = N // nb
    return pl.pallas_call(
        _combine1_body,
        grid=(nb,),
        in_specs=[
            pl.BlockSpec((rows, D), lambda i: (i, 0)),
            pl.BlockSpec((rows, 1), lambda i: (i, 0)),
            pl.BlockSpec((rows, D), lambda i: (i, 0)),
            pl.BlockSpec((D, D), lambda i: (0, 0)),
            pl.BlockSpec((D, D), lambda i: (0, 0)),
            pl.BlockSpec((1, D), lambda i: (0, 0)),
        ],
        out_specs=[
            pl.BlockSpec((rows, D), lambda i: (i, 0)),
            pl.BlockSpec((rows, D), lambda i: (i, 0)),
        ],
        out_shape=[
            jax.ShapeDtypeStruct((N, D), jnp.float32),
            jax.ShapeDtypeStruct((N, D), jnp.float32),
        ],
    )(agg, cnt, r, wl, wr, b)


def _final_body(agg_ref, cnt_ref, r_ref, o_ref):
    mean = agg_ref[...] / jnp.maximum(cnt_ref[...], 1.0)
    h = mean + r_ref[...]
    m = jnp.max(h, axis=1, keepdims=True)
    ex = jnp.exp(h - m)
    s = jnp.sum(ex, axis=1, keepdims=True)
    o_ref[...] = h - m - jnp.log(s)


def _final(agg, cnt, r):
    nb = 10
    rows = N // nb
    return pl.pallas_call(
        _final_body,
        grid=(nb,),
        in_specs=[
            pl.BlockSpec((rows, D), lambda i: (i, 0)),
            pl.BlockSpec((rows, 1), lambda i: (i, 0)),
            pl.BlockSpec((rows, D), lambda i: (i, 0)),
        ],
        out_specs=pl.BlockSpec((rows, D), lambda i: (i, 0)),
        out_shape=jax.ShapeDtypeStruct((N, D), jnp.float32),
    )(agg, cnt, r)


def _assemble(acc):
    agg = jnp.concatenate([acc[0, :BASEN], acc[1, :BASEN]], axis=0)
    c = acc[:, BASEN:BASEN + CROWS, :].reshape(2, CROWS * D)[:, :BASEN]
    cnt = c.reshape(2 * BASEN, 1)
    return agg, cnt


def kernel(x, edge_index, Wl1, bl1, Wr1, Wl2, bl2, Wr2):
    pad = EPAD - E

    src_p = jnp.concatenate([edge_index[0], jnp.zeros((pad,), jnp.int32)])
    dst_p = jnp.concatenate([edge_index[1], jnp.full((pad,), N, jnp.int32)])
    b1 = bl1.reshape(1, D)
    b2 = bl2.reshape(1, D)

    y1, r1 = _mm_pre(x, Wl1, Wr1, b1)
    acc1 = _sc_pass_cnt(y1, src_p, dst_p)
    agg1, cnt = _assemble(acc1)
    y2, r2 = _combine1(agg1, cnt, r1, Wl2, Wr2, b2)
    acc2 = _sc_pass(y2, src_p, dst_p)
    agg2 = jnp.concatenate([acc2[0, :BASEN], acc2[1, :BASEN]], axis=0)
    out = _final(agg2, cnt, r2)
    return out
